# baseline (speedup 1.0000x reference)
import functools

import jax
import jax.numpy as jnp
from jax import lax
from jax.experimental import pallas as pl
from jax.experimental.pallas import tpu as pltpu
from jax.experimental.pallas import tpu_sc as plsc

R = 128
N = 32768
K = 64
L = 16
CH = 128
NCH = N // CH
NSUP = 16
BIG = 1 << 30
NEG = float("-inf")


def _permute(v, idx):
    dn = lax.GatherDimensionNumbers(
        offset_dims=(), collapsed_slice_dims=(0,), start_index_map=(0,))
    return lax.gather(v, idx[:, None], dn, slice_sizes=(1,),
                      mode=lax.GatherScatterMode.PROMISE_IN_BOUNDS)


def _bmax(v):
    iota = lax.iota(jnp.int32, L)
    for sh in (1, 2, 4, 8):
        v = jnp.maximum(v, _permute(v, jnp.bitwise_xor(iota, sh)))
    return v


def _bmin(v):
    iota = lax.iota(jnp.int32, L)
    for sh in (1, 2, 4, 8):
        v = jnp.minimum(v, _permute(v, jnp.bitwise_xor(iota, sh)))
    return v


def _row_topk(row_v, cmax_v, vals_v, poss_v, outb_v):
    iota = lax.iota(jnp.int32, L)
    neg_vec = jnp.full((L,), NEG, jnp.float32)

    def sup_body(s, U):
        accv = neg_vec
        for j in range(L):
            base = (s * L + j) * CH
            vs = [row_v[pl.ds(base + k * L, L)] for k in range(CH // L)]
            while len(vs) > 1:
                vs = [jnp.maximum(vs[i], vs[i + 1])
                      for i in range(0, len(vs), 2)]
            accv = jnp.where(iota == j, jnp.max(vs[0]), accv)
        cmax_v[pl.ds(s * L, L)] = accv
        return jnp.where(iota == s, jnp.max(accv), U)

    U = lax.fori_loop(0, NSUP, sup_body, neg_vec)

    def ext_body(i, U):
        m = _bmax(U)
        s = _bmin(jnp.where(U == m, iota, BIG))
        t = plsc.load_gather(cmax_v, [s * L + iota])
        c = _bmin(jnp.where(t == m, s * L + iota, BIG))
        base = c * CH
        idxs = [base + iota + k * L for k in range(CH // L)]
        vs = [plsc.load_gather(row_v, [idxs[k]]) for k in range(CH // L)]
        pos = jnp.full((L,), BIG, jnp.int32)
        for k in range(CH // L):
            pos = jnp.minimum(pos, jnp.where(vs[k] == m, idxs[k], BIG))
        pos = _bmin(pos)
        nm = neg_vec
        for k in range(CH // L):
            w = jnp.where(idxs[k] == pos, NEG, vs[k])
            plsc.store_scatter(row_v, [idxs[k]], w)
            nm = jnp.maximum(nm, w)
        t2 = jnp.where(s * L + iota == c, _bmax(nm), t)
        plsc.store_scatter(cmax_v, [s * L + iota], t2)
        rec = jnp.broadcast_to(i, (L,))
        lane0 = iota == 0
        plsc.store_scatter(vals_v, [rec], m, mask=lane0)
        plsc.store_scatter(poss_v, [rec], pos, mask=lane0)
        return jnp.where(iota == s, _bmax(t2), U)

    U = lax.fori_loop(0, K, ext_body, U)

    pv = [poss_v[pl.ds(a * L, L)] for a in range(K // L)]
    maxp = pv[0][0]
    Rv = [jnp.bitwise_and(p - maxp, N - 1) for p in pv]
    Vv = [vals_v[pl.ds(a * L, L)] for a in range(K // L)]

    rot1 = jnp.bitwise_and(iota + 1, L - 1)

    def rank_body(_, carry):
        Ks, rbs = carry
        Ks = tuple(
            Ka + sum((rb < Ra).astype(jnp.int32) for rb in rbs)
            for Ka, Ra in zip(Ks, Rv))
        rbs = tuple(_permute(rb, rot1) for rb in rbs)
        return Ks, rbs

    Ks, _ = lax.fori_loop(
        0, L, rank_body,
        (tuple(jnp.zeros((L,), jnp.int32) for _ in range(K // L)), tuple(Rv)))
    for a in range(K // L):
        plsc.store_scatter(outb_v, [Ks[a]], Vv[a])
    return U


NUM_CORES = 2
NUM_SUBCORES = 16


def kernel(tens):
    nw = NUM_CORES * NUM_SUBCORES
    rows_per = R // nw
    mesh = plsc.VectorSubcoreMesh(
        core_axis_name="c", subcore_axis_name="s",
        num_cores=NUM_CORES, num_subcores=NUM_SUBCORES)

    @functools.partial(
        pl.kernel,
        mesh=mesh,
        out_type=jax.ShapeDtypeStruct((R, K), jnp.float32),
        scratch_types=[
            pltpu.VMEM((N,), jnp.float32),
            pltpu.VMEM((N,), jnp.float32),
            pltpu.VMEM((NCH,), jnp.float32),
            pltpu.VMEM((K,), jnp.float32),
            pltpu.VMEM((K,), jnp.int32),
            pltpu.VMEM((K,), jnp.float32),
            pltpu.SemaphoreType.DMA,
            pltpu.SemaphoreType.DMA,
        ],
        compiler_params=pltpu.CompilerParams(needs_layout_passes=False),
    )
    def run(tens_hbm, out_hbm, row_a, row_b, cmax_v, vals_v, poss_v, outb_v,
            sem_a, sem_b):
        wid = lax.axis_index("s") * NUM_CORES + lax.axis_index("c")
        base = wid * rows_per
        bufs = (row_a, row_b)
        sems = (sem_a, sem_b)

        cps = [None, None]
        cps[0] = pltpu.async_copy(tens_hbm.at[base], bufs[0], sems[0])
        for j in range(rows_per):
            if j + 1 < rows_per:
                nb = (j + 1) % 2
                cps[nb] = pltpu.async_copy(
                    tens_hbm.at[base + j + 1], bufs[nb], sems[nb])
            cb = j % 2
            cps[cb].wait()
            _row_topk(bufs[cb], cmax_v, vals_v, poss_v, outb_v)
            pltpu.sync_copy(outb_v, out_hbm.at[base + j])

    return run(tens)


# speedup vs baseline: 1.0489x; 1.0489x over previous
import functools

import jax
import jax.numpy as jnp
from jax import lax
from jax.experimental import pallas as pl
from jax.experimental.pallas import tpu as pltpu
from jax.experimental.pallas import tpu_sc as plsc

R = 128
N = 32768
K = 64
L = 16
CH = 128
NCH = N // CH
NSUP = 16
BIG = 1 << 30
NEG = float("-inf")


def _permute(v, idx):
    dn = lax.GatherDimensionNumbers(
        offset_dims=(), collapsed_slice_dims=(0,), start_index_map=(0,))
    return lax.gather(v, idx[:, None], dn, slice_sizes=(1,),
                      mode=lax.GatherScatterMode.PROMISE_IN_BOUNDS)


def _bmax(v):
    iota = lax.iota(jnp.int32, L)
    for sh in (1, 2, 4, 8):
        v = jnp.maximum(v, _permute(v, jnp.bitwise_xor(iota, sh)))
    return v


def _bmin(v):
    iota = lax.iota(jnp.int32, L)
    for sh in (1, 2, 4, 8):
        v = jnp.minimum(v, _permute(v, jnp.bitwise_xor(iota, sh)))
    return v


def _row_topk(row_v, cmax_v, vals_v, poss_v, outb_v):
    iota = lax.iota(jnp.int32, L)
    neg_vec = jnp.full((L,), NEG, jnp.float32)

    def sup_body(s, U):
        def ch_body(j, accv):
            base = (s * L + j) * CH
            m = row_v[pl.ds(base, L)]
            for k in range(1, CH // L):
                m = jnp.maximum(m, row_v[pl.ds(base + k * L, L)])
            return jnp.where(iota == j, jnp.max(m), accv)

        accv = lax.fori_loop(0, L, ch_body, neg_vec, unroll=4)
        cmax_v[pl.ds(s * L, L)] = accv
        return jnp.where(iota == s, jnp.max(accv), U)

    U = lax.fori_loop(0, NSUP, sup_body, neg_vec)

    def ext_body(i, U):
        m = _bmax(U)
        s = _bmin(jnp.where(U == m, iota, BIG))
        t = plsc.load_gather(cmax_v, [s * L + iota])
        c = _bmin(jnp.where(t == m, s * L + iota, BIG))
        base = c * CH
        idxs = [base + iota + k * L for k in range(CH // L)]
        vs = [plsc.load_gather(row_v, [idxs[k]]) for k in range(CH // L)]
        pos = jnp.full((L,), BIG, jnp.int32)
        for k in range(CH // L):
            pos = jnp.minimum(pos, jnp.where(vs[k] == m, idxs[k], BIG))
        pos = _bmin(pos)
        nm = neg_vec
        for k in range(CH // L):
            w = jnp.where(idxs[k] == pos, NEG, vs[k])
            plsc.store_scatter(row_v, [idxs[k]], w)
            nm = jnp.maximum(nm, w)
        t2 = jnp.where(s * L + iota == c, _bmax(nm), t)
        plsc.store_scatter(cmax_v, [s * L + iota], t2)
        rec = jnp.broadcast_to(i, (L,))
        lane0 = iota == 0
        plsc.store_scatter(vals_v, [rec], m, mask=lane0)
        plsc.store_scatter(poss_v, [rec], pos, mask=lane0)
        return jnp.where(iota == s, _bmax(t2), U)

    U = lax.fori_loop(0, K, ext_body, U)

    pv = [poss_v[pl.ds(a * L, L)] for a in range(K // L)]
    maxp = pv[0][0]
    Rv = [jnp.bitwise_and(p - maxp, N - 1) for p in pv]
    Vv = [vals_v[pl.ds(a * L, L)] for a in range(K // L)]

    rot1 = jnp.bitwise_and(iota + 1, L - 1)

    def rank_body(_, carry):
        Ks, rbs = carry
        Ks = tuple(
            Ka + sum((rb < Ra).astype(jnp.int32) for rb in rbs)
            for Ka, Ra in zip(Ks, Rv))
        rbs = tuple(_permute(rb, rot1) for rb in rbs)
        return Ks, rbs

    Ks, _ = lax.fori_loop(
        0, L, rank_body,
        (tuple(jnp.zeros((L,), jnp.int32) for _ in range(K // L)), tuple(Rv)))
    for a in range(K // L):
        plsc.store_scatter(outb_v, [Ks[a]], Vv[a])
    return U


NUM_CORES = 2
NUM_SUBCORES = 16


def kernel(tens):
    nw = NUM_CORES * NUM_SUBCORES
    rows_per = R // nw
    mesh = plsc.VectorSubcoreMesh(
        core_axis_name="c", subcore_axis_name="s",
        num_cores=NUM_CORES, num_subcores=NUM_SUBCORES)

    @functools.partial(
        pl.kernel,
        mesh=mesh,
        out_type=jax.ShapeDtypeStruct((R, K), jnp.float32),
        scratch_types=[
            pltpu.VMEM((N,), jnp.float32),
            pltpu.VMEM((N,), jnp.float32),
            pltpu.VMEM((NCH,), jnp.float32),
            pltpu.VMEM((K,), jnp.float32),
            pltpu.VMEM((K,), jnp.int32),
            pltpu.VMEM((K,), jnp.float32),
            pltpu.SemaphoreType.DMA,
            pltpu.SemaphoreType.DMA,
        ],
        compiler_params=pltpu.CompilerParams(needs_layout_passes=False),
    )
    def run(tens_hbm, out_hbm, row_a, row_b, cmax_v, vals_v, poss_v, outb_v,
            sem_a, sem_b):
        wid = lax.axis_index("s") * NUM_CORES + lax.axis_index("c")
        base = wid * rows_per
        bufs = (row_a, row_b)
        sems = (sem_a, sem_b)

        cps = [None, None]
        cps[0] = pltpu.async_copy(tens_hbm.at[base], bufs[0], sems[0])
        for j in range(rows_per):
            if j + 1 < rows_per:
                nb = (j + 1) % 2
                cps[nb] = pltpu.async_copy(
                    tens_hbm.at[base + j + 1], bufs[nb], sems[nb])
            cb = j % 2
            cps[cb].wait()
            _row_topk(bufs[cb], cmax_v, vals_v, poss_v, outb_v)
            pltpu.sync_copy(outb_v, out_hbm.at[base + j])

    return run(tens)
